# baseline (speedup 1.0000x reference)
import functools

import jax
import jax.numpy as jnp
from jax.experimental import pallas as pl
from jax.experimental.pallas import tpu as pltpu

LB_POS = 0.9
LB_NEG = 0.005
LB_IGNORE = 255

B = 1024
C = 100000
BC = 1024
K = (C + BC - 1) // BC


def _body(x_ref, lab_ref, out_ref, m_ref, s_ref, t_ref, g_ref):
    k = pl.program_id(0)

    @pl.when(k == 0)
    def _init():
        m_ref[...] = jnp.full((B, 1), -jnp.inf, jnp.float32)
        s_ref[...] = jnp.zeros((B, 1), jnp.float32)
        t_ref[...] = jnp.zeros((B, 1), jnp.float32)
        g_ref[...] = jnp.zeros((B, 1), jnp.float32)

    x = x_ref[...]
    ids = jax.lax.broadcasted_iota(jnp.int32, (1, BC), 1) + k * BC
    valid = ids < C

    xm = jnp.where(valid, x, -jnp.inf)
    m_prev = m_ref[...]
    m_new = jnp.maximum(m_prev, jnp.max(xm, axis=1, keepdims=True))
    s_blk = jnp.sum(jnp.exp(xm - m_new), axis=1, keepdims=True)
    s_ref[...] = s_ref[...] * jnp.exp(m_prev - m_new) + s_blk
    m_ref[...] = m_new

    t_ref[...] += jnp.sum(jnp.where(valid, x, 0.0), axis=1, keepdims=True)

    lab = lab_ref[...]
    eq = ids == lab
    g_ref[...] += jnp.sum(jnp.where(eq, x, 0.0), axis=1, keepdims=True)

    @pl.when(k == K - 1)
    def _fin():
        lse = m_ref[...] + jnp.log(s_ref[...])
        ign = lab == LB_IGNORE
        contrib = LB_NEG * (t_ref[...] - C * lse) + (LB_POS - LB_NEG) * (
            g_ref[...] - lse
        )
        contrib = jnp.where(ign, 0.0, contrib)
        n_valid = jnp.sum(jnp.where(ign, 0.0, 1.0))
        out_ref[...] = (-jnp.sum(contrib) / n_valid).reshape(1, 1)


@jax.jit
def kernel(logits, label):
    out = pl.pallas_call(
        _body,
        grid=(K,),
        in_specs=[
            pl.BlockSpec((B, BC), lambda k: (0, k)),
            pl.BlockSpec((B, 1), lambda k: (0, 0)),
        ],
        out_specs=pl.BlockSpec((1, 1), lambda k: (0, 0)),
        out_shape=jax.ShapeDtypeStruct((1, 1), jnp.float32),
        scratch_shapes=[pltpu.VMEM((B, 1), jnp.float32)] * 4,
        compiler_params=pltpu.CompilerParams(
            dimension_semantics=("arbitrary",),
        ),
    )(logits, label.reshape(B, 1))
    return out[0, 0]


# speedup vs baseline: 1.0595x; 1.0595x over previous
import functools

import jax
import jax.numpy as jnp
from jax.experimental import pallas as pl
from jax.experimental.pallas import tpu as pltpu

LB_POS = 0.9
LB_NEG = 0.005
LB_IGNORE = 255

B = 1024
C = 100000
BC = 2048
K = (C + BC - 1) // BC


def _body(x_ref, lab_ref, out_ref, m_ref, s_ref, t_ref, g_ref):
    k = pl.program_id(0)

    @pl.when(k == 0)
    def _init():
        m_ref[...] = jnp.full((B, 1), -jnp.inf, jnp.float32)
        s_ref[...] = jnp.zeros((B, 1), jnp.float32)
        t_ref[...] = jnp.zeros((B, 1), jnp.float32)
        g_ref[...] = jnp.zeros((B, 1), jnp.float32)

    x = x_ref[...]
    ids = jax.lax.broadcasted_iota(jnp.int32, (1, BC), 1) + k * BC
    valid = ids < C

    xm = jnp.where(valid, x, -jnp.inf)
    m_prev = m_ref[...]
    m_new = jnp.maximum(m_prev, jnp.max(xm, axis=1, keepdims=True))
    s_blk = jnp.sum(jnp.exp(xm - m_new), axis=1, keepdims=True)
    s_ref[...] = s_ref[...] * jnp.exp(m_prev - m_new) + s_blk
    m_ref[...] = m_new

    t_ref[...] += jnp.sum(jnp.where(valid, x, 0.0), axis=1, keepdims=True)

    lab = lab_ref[...]
    eq = ids == lab
    g_ref[...] += jnp.sum(jnp.where(eq, x, 0.0), axis=1, keepdims=True)

    @pl.when(k == K - 1)
    def _fin():
        lse = m_ref[...] + jnp.log(s_ref[...])
        ign = lab == LB_IGNORE
        contrib = LB_NEG * (t_ref[...] - C * lse) + (LB_POS - LB_NEG) * (
            g_ref[...] - lse
        )
        contrib = jnp.where(ign, 0.0, contrib)
        n_valid = jnp.sum(jnp.where(ign, 0.0, 1.0))
        out_ref[...] = (-jnp.sum(contrib) / n_valid).reshape(1, 1)


@jax.jit
def kernel(logits, label):
    out = pl.pallas_call(
        _body,
        grid=(K,),
        in_specs=[
            pl.BlockSpec((B, BC), lambda k: (0, k)),
            pl.BlockSpec((B, 1), lambda k: (0, 0)),
        ],
        out_specs=pl.BlockSpec((1, 1), lambda k: (0, 0)),
        out_shape=jax.ShapeDtypeStruct((1, 1), jnp.float32),
        scratch_shapes=[pltpu.VMEM((B, 1), jnp.float32)] * 4,
        compiler_params=pltpu.CompilerParams(
            dimension_semantics=("arbitrary",),
        ),
    )(logits, label.reshape(B, 1))
    return out[0, 0]
